# Initial kernel scaffold; baseline (speedup 1.0000x reference)
import functools

import jax
import jax.numpy as jnp
from jax.experimental import pallas as pl
from jax.experimental.pallas import tpu as pltpu

N = 10000
E = 320000
IN = 128
HID = 128
EMB = 128
H = 8

BN = 500


def _k1_body(x_ref, w_ref, acat_ref, h_ref, a_ref):
    xb = x_ref[...]
    hb = jnp.dot(xb, w_ref[...], preferred_element_type=jnp.float32)
    h_ref[...] = hb
    a_ref[...] = jnp.dot(hb, acat_ref[...], preferred_element_type=jnp.float32)


def _k1(x, W, acat):
    return pl.pallas_call(
        _k1_body,
        grid=(N // BN,),
        in_specs=[
            pl.BlockSpec((BN, IN), lambda i: (i, 0)),
            pl.BlockSpec((IN, H * HID), lambda i: (0, 0)),
            pl.BlockSpec((H * HID, 128), lambda i: (0, 0)),
        ],
        out_specs=[
            pl.BlockSpec((BN, H * HID), lambda i: (i, 0)),
            pl.BlockSpec((BN, 128), lambda i: (i, 0)),
        ],
        out_shape=[
            jax.ShapeDtypeStruct((N, H * HID), jnp.float32),
            jax.ShapeDtypeStruct((N, 128), jnp.float32),
        ],
    )(x, W, acat)


def _k6_body(g_ref, bias_ref, wemb_ref, bemb_ref, wd1_ref, bd1_ref, wd2_ref,
             bd2_ref, xr_ref, z_ref):
    g = g_ref[...]
    hg = jax.nn.relu(g + bias_ref[...])
    z = jnp.dot(hg, wemb_ref[...], preferred_element_type=jnp.float32) + bemb_ref[...]
    d = jax.nn.relu(jnp.dot(z, wd1_ref[...], preferred_element_type=jnp.float32) + bd1_ref[...])
    xr = jnp.dot(d, wd2_ref[...], preferred_element_type=jnp.float32) + bd2_ref[...]
    z_ref[...] = z
    xr_ref[...] = xr


def _k6(gatout, bias_gat, W_emb, b_emb, W_d1, b_d1, W_d2, b_d2):
    b2 = bias_gat.reshape(1, H * HID)
    be = b_emb.reshape(1, EMB)
    b1 = b_d1.reshape(1, HID)
    bd = b_d2.reshape(1, IN)
    return pl.pallas_call(
        _k6_body,
        grid=(N // BN,),
        in_specs=[
            pl.BlockSpec((BN, H * HID), lambda i: (i, 0)),
            pl.BlockSpec((1, H * HID), lambda i: (0, 0)),
            pl.BlockSpec((H * HID, EMB), lambda i: (0, 0)),
            pl.BlockSpec((1, EMB), lambda i: (0, 0)),
            pl.BlockSpec((EMB, HID), lambda i: (0, 0)),
            pl.BlockSpec((1, HID), lambda i: (0, 0)),
            pl.BlockSpec((HID, IN), lambda i: (0, 0)),
            pl.BlockSpec((1, IN), lambda i: (0, 0)),
        ],
        out_specs=[
            pl.BlockSpec((BN, IN), lambda i: (i, 0)),
            pl.BlockSpec((BN, EMB), lambda i: (i, 0)),
        ],
        out_shape=[
            jax.ShapeDtypeStruct((N, IN), jnp.float32),
            jax.ShapeDtypeStruct((N, EMB), jnp.float32),
        ],
    )(gatout, b2, W_emb, be, W_d1, b1, W_d2, bd)


def kernel(x, edge_index, edge_weight, W, att_src, att_dst, bias_gat, W_emb,
           b_emb, W_d1, b_d1, W_d2, b_d2):
    acat = jnp.zeros((H, HID, 128), jnp.float32)
    hh = jnp.arange(H)
    acat = acat.at[hh, :, hh].set(att_src)
    acat = acat.at[hh, :, hh + H].set(att_dst)
    acat = acat.reshape(H * HID, 128)

    h_flat, a_all = _k1(x, W, acat)
    a_src = a_all[:, :H]
    a_dst = a_all[:, H:2 * H]

    src = edge_index[0]
    dst = edge_index[1]
    alpha = jax.nn.leaky_relu(a_src[src] + a_dst[dst], negative_slope=0.2)
    ex = jnp.exp(alpha) * edge_weight[:, None]
    denom = jax.ops.segment_sum(ex, dst, num_segments=N)
    c = ex / (denom[dst] + 1e-16)
    h3 = h_flat.reshape(N, H, HID)
    msg = h3[src] * c[:, :, None]
    gatout = jax.ops.segment_sum(msg, dst, num_segments=N).reshape(N, H * HID)

    xr, z = _k6(gatout, bias_gat, W_emb, b_emb, W_d1, b_d1, W_d2, b_d2)
    return (xr, z)


# speedup vs baseline: 1.0504x; 1.0504x over previous
import functools

import jax
import jax.numpy as jnp
from jax.experimental import pallas as pl
from jax.experimental.pallas import tpu as pltpu

N = 10000
E = 320000
IN = 128
HID = 128
EMB = 128
H = 8

BN = 1000


def _k1_body(x_ref, w_ref, acat_ref, h_ref, a_ref):
    xb = x_ref[...]
    hb = jnp.dot(xb, w_ref[...], preferred_element_type=jnp.float32)
    h_ref[...] = hb
    a_ref[...] = jnp.dot(hb, acat_ref[...], preferred_element_type=jnp.float32)


def _k1(x, W, acat):
    return pl.pallas_call(
        _k1_body,
        grid=(N // BN,),
        in_specs=[
            pl.BlockSpec((BN, IN), lambda i: (i, 0)),
            pl.BlockSpec((IN, H * HID), lambda i: (0, 0)),
            pl.BlockSpec((H * HID, 128), lambda i: (0, 0)),
        ],
        out_specs=[
            pl.BlockSpec((BN, H * HID), lambda i: (i, 0)),
            pl.BlockSpec((BN, 128), lambda i: (i, 0)),
        ],
        out_shape=[
            jax.ShapeDtypeStruct((N, H * HID), jnp.float32),
            jax.ShapeDtypeStruct((N, 128), jnp.float32),
        ],
    )(x, W, acat)


def _k6_body(g_ref, bias_ref, wemb_ref, bemb_ref, wd1_ref, bd1_ref, wd2_ref,
             bd2_ref, xr_ref, z_ref):
    g = g_ref[...]
    hg = jax.nn.relu(g + bias_ref[...])
    z = jnp.dot(hg, wemb_ref[...], preferred_element_type=jnp.float32) + bemb_ref[...]
    d = jax.nn.relu(jnp.dot(z, wd1_ref[...], preferred_element_type=jnp.float32) + bd1_ref[...])
    xr = jnp.dot(d, wd2_ref[...], preferred_element_type=jnp.float32) + bd2_ref[...]
    z_ref[...] = z
    xr_ref[...] = xr


def _k6(gatout, bias_gat, W_emb, b_emb, W_d1, b_d1, W_d2, b_d2):
    b2 = bias_gat.reshape(1, H * HID)
    be = b_emb.reshape(1, EMB)
    b1 = b_d1.reshape(1, HID)
    bd = b_d2.reshape(1, IN)
    return pl.pallas_call(
        _k6_body,
        grid=(N // BN,),
        in_specs=[
            pl.BlockSpec((BN, H * HID), lambda i: (i, 0)),
            pl.BlockSpec((1, H * HID), lambda i: (0, 0)),
            pl.BlockSpec((H * HID, EMB), lambda i: (0, 0)),
            pl.BlockSpec((1, EMB), lambda i: (0, 0)),
            pl.BlockSpec((EMB, HID), lambda i: (0, 0)),
            pl.BlockSpec((1, HID), lambda i: (0, 0)),
            pl.BlockSpec((HID, IN), lambda i: (0, 0)),
            pl.BlockSpec((1, IN), lambda i: (0, 0)),
        ],
        out_specs=[
            pl.BlockSpec((BN, IN), lambda i: (i, 0)),
            pl.BlockSpec((BN, EMB), lambda i: (i, 0)),
        ],
        out_shape=[
            jax.ShapeDtypeStruct((N, IN), jnp.float32),
            jax.ShapeDtypeStruct((N, EMB), jnp.float32),
        ],
    )(gatout, b2, W_emb, be, W_d1, b1, W_d2, bd)


def kernel(x, edge_index, edge_weight, W, att_src, att_dst, bias_gat, W_emb,
           b_emb, W_d1, b_d1, W_d2, b_d2):
    acat = jnp.zeros((H, HID, 128), jnp.float32)
    hh = jnp.arange(H)
    acat = acat.at[hh, :, hh].set(att_src)
    acat = acat.at[hh, :, hh + H].set(att_dst)
    acat = acat.reshape(H * HID, 128)

    h_flat, a_all = _k1(x, W, acat)
    a_src = a_all[:, :H]
    a_dst = a_all[:, H:2 * H]

    src = edge_index[0]
    dst = edge_index[1]
    alpha = jax.nn.leaky_relu(a_src[src] + a_dst[dst], negative_slope=0.2)
    ex = jnp.exp(alpha)
    denom = jax.ops.segment_sum(ex, dst, num_segments=N)
    c = ex * edge_weight[:, None] / (denom[dst] + 1e-16)
    h3 = h_flat.reshape(N, H, HID)
    msg = h3[src] * c[:, :, None]
    gatout = jax.ops.segment_sum(msg, dst, num_segments=N).reshape(N, H * HID)

    xr, z = _k6(gatout, bias_gat, W_emb, b_emb, W_d1, b_d1, W_d2, b_d2)
    return (xr, z)
